# baseline (speedup 1.0000x reference)
import functools

import jax
import jax.numpy as jnp
import numpy as np
from jax import lax
from jax.experimental import pallas as pl
from jax.experimental.pallas import tpu as pltpu
from jax.experimental.pallas import tpu_sc as plsc

PDIM = 2
SEQ_LEN = 6
LATENT = 128
NSTEPS = 5
RADIUS = 0.032
NTYPES = 9
CLAMP = 1.0
VEL_STD = 0.0017
ACC_STD = 3e-4
MAX_EDGES = 600000

N_PAD = 10240
E_PAD = 606208
BN = 512
BE = 1024
SC_CHUNK = 128

_f32 = jnp.float32


def _radius_edges(pos, nper):
    n_total = pos.shape[0]
    r2 = (RADIUS + 1e-8) ** 2
    thr = np.float32(r2)
    if float(thr) > r2:
        thr = np.nextafter(thr, np.float32(0.0))
    csum = jnp.cumsum(nper)
    ex = jnp.searchsorted(csum, jnp.arange(n_total), side="right")
    valid = ex < nper.shape[0]
    rows = []
    for i0 in range(0, n_total, 2048):
        blk = pos[i0:i0 + 2048]
        d2 = ((blk[:, None, :] - pos[None, :, :]) ** 2).sum(-1)
        m = (d2 <= thr) & (ex[i0:i0 + 2048, None] == ex[None, :]) & valid[i0:i0 + 2048, None]
        rows.append(m)
    mask = jnp.concatenate(rows, axis=0)
    receivers, senders = jnp.nonzero(mask, size=MAX_EDGES, fill_value=n_total)
    return receivers.astype(jnp.int32), senders.astype(jnp.int32)


def _sc_gather_pair(table_r, table_s, idx_r, idx_s):
    E = idx_r.shape[0]
    D = table_r.shape[1]
    info = plsc.get_sparse_core_info()
    nc, ns = info.num_cores, info.num_subcores
    nw = nc * ns
    bw = E // nw
    nch = bw // SC_CHUNK
    mesh = plsc.VectorSubcoreMesh(core_axis_name="c", subcore_axis_name="s")

    @functools.partial(
        pl.kernel, mesh=mesh,
        out_type=(jax.ShapeDtypeStruct((E, D), _f32),
                  jax.ShapeDtypeStruct((E, D), _f32)),
        scratch_types=[
            pltpu.VMEM((SC_CHUNK,), jnp.int32),
            pltpu.VMEM((SC_CHUNK,), jnp.int32),
            pltpu.VMEM((SC_CHUNK, D), _f32),
            pltpu.VMEM((SC_CHUNK, D), _f32),
            pltpu.SemaphoreType.DMA,
            pltpu.SemaphoreType.DMA,
        ],
    )
    def k(t_r, t_s, i_r, i_s, o_r, o_s, iv_r, iv_s, rv_r, rv_s, sem_r, sem_s):
        wid = lax.axis_index("s") * nc + lax.axis_index("c")
        base = wid * bw

        def body(j, carry):
            off = base + j * SC_CHUNK
            pltpu.sync_copy(i_r.at[pl.ds(off, SC_CHUNK)], iv_r)
            pltpu.sync_copy(i_s.at[pl.ds(off, SC_CHUNK)], iv_s)
            c_r = pltpu.async_copy(t_r.at[iv_r], rv_r, sem_r)
            c_s = pltpu.async_copy(t_s.at[iv_s], rv_s, sem_s)
            c_r.wait()
            c_s.wait()
            pltpu.sync_copy(rv_r, o_r.at[pl.ds(off, SC_CHUNK)])
            pltpu.sync_copy(rv_s, o_s.at[pl.ds(off, SC_CHUNK)])
            return carry

        lax.fori_loop(0, nch, body, 0)

    return k(table_r, table_s, idx_r, idx_s)


def _sc_scatter_add(rows, idx, zeros):
    E = idx.shape[0]
    info = plsc.get_sparse_core_info()
    nc, ns = info.num_cores, info.num_subcores
    nw = nc * ns
    bw = E // nw
    nch = bw // SC_CHUNK
    rps = N_PAD // ns
    mesh = plsc.VectorSubcoreMesh(core_axis_name="c", subcore_axis_name="s")

    @functools.partial(
        pl.kernel, mesh=mesh,
        out_type=jax.ShapeDtypeStruct((nc, N_PAD, LATENT), _f32),
        scratch_types=[
            pltpu.VMEM((SC_CHUNK,), jnp.int32),
            pltpu.VMEM((SC_CHUNK, LATENT), _f32),
            pltpu.VMEM_SHARED((N_PAD, LATENT), _f32),
        ],
    )
    def k(rows_h, idx_h, zeros_h, out_h, iv, rv, acc):
        cid = lax.axis_index("c")
        sid = lax.axis_index("s")
        wid = sid * nc + cid
        base = wid * bw
        pltpu.sync_copy(zeros_h.at[pl.ds(sid * rps, rps)],
                        acc.at[pl.ds(sid * rps, rps)])
        plsc.subcore_barrier()

        def body(j, carry):
            off = base + j * SC_CHUNK
            pltpu.sync_copy(idx_h.at[pl.ds(off, SC_CHUNK)], iv)
            pltpu.sync_copy(rows_h.at[pl.ds(off, SC_CHUNK)], rv)
            pltpu.sync_copy(rv, acc.at[iv], add=True)
            return carry

        lax.fori_loop(0, nch, body, 0)
        plsc.subcore_barrier()
        pltpu.sync_copy(acc.at[pl.ds(sid * rps, rps)],
                        out_h.at[cid, pl.ds(sid * rps, rps)])

    return k(rows, idx, zeros)


def _ln(u, g, b):
    mu = jnp.mean(u, axis=-1, keepdims=True)
    var = jnp.mean((u - mu) ** 2, axis=-1, keepdims=True)
    return (u - mu) * lax.rsqrt(var + 1e-5) * g + b


def _dot(a, b):
    return jnp.dot(a, b, preferred_element_type=_f32)


_W = pl.BlockSpec((LATENT, LATENT), lambda i: (0, 0))
_B = pl.BlockSpec((1, LATENT), lambda i: (0, 0))


def _eblk(d=LATENT):
    return pl.BlockSpec((BE, d), lambda i: (i, 0))


def _nblk(d=LATENT):
    return pl.BlockSpec((BN, d), lambda i: (i, 0))


def _node_enc_call(seq12, ty, temb, w1v, w1bd, w1e, b1, w2, b2, w3, b3,
                   lg, lb, wra, wrb):
    grid = (N_PAD // BN,)

    def kfn(seq_r, ty_r, temb_r, w1v_r, w1bd_r, w1e_r, b1_r, w2_r, b2_r,
            w3_r, b3_r, lg_r, lb_r, wra_r, wrb_r, x_o, xr_o, xs_o):
        a = seq_r[...]
        v = (a[:, 2:12] - a[:, 0:10]) / VEL_STD
        mr = a[:, 10:12]
        bd = jnp.clip(
            jnp.concatenate([mr / RADIUS, (1.0 - mr) / RADIUS], axis=1),
            -CLAMP, CLAMP)
        t = ty_r[...]
        lanes = lax.broadcasted_iota(jnp.int32, (BN, 16), 1)
        oh = (lanes == t).astype(_f32)
        emb = _dot(oh, temb_r[...])
        h = _dot(v, w1v_r[...]) + _dot(bd, w1bd_r[...]) + _dot(emb, w1e_r[...]) + b1_r[...]
        h = jnp.maximum(h, 0.0)
        h = jnp.maximum(_dot(h, w2_r[...]) + b2_r[...], 0.0)
        u = _dot(h, w3_r[...]) + b3_r[...]
        x0 = _ln(u, lg_r[...], lb_r[...])
        x_o[...] = x0
        xr_o[...] = _dot(x0, wra_r[...])
        xs_o[...] = _dot(x0, wrb_r[...])

    return pl.pallas_call(
        kfn, grid=grid,
        in_specs=[
            _nblk(12),
            pl.BlockSpec((BN, 1), lambda i: (i, 0)),
            pl.BlockSpec((16, 16), lambda i: (0, 0)),
            pl.BlockSpec((10, LATENT), lambda i: (0, 0)),
            pl.BlockSpec((4, LATENT), lambda i: (0, 0)),
            pl.BlockSpec((16, LATENT), lambda i: (0, 0)),
            _B, _W, _B, _W, _B, _B, _B, _W, _W,
        ],
        out_specs=[_nblk(), _nblk(), _nblk()],
        out_shape=[jax.ShapeDtypeStruct((N_PAD, LATENT), _f32)] * 3,
    )(seq12, ty, temb, w1v, w1bd, w1e, b1, w2, b2, w3, b3, lg, lb, wra, wrb)


def _edge_step0_call(pr8, ps8, gr, gs, er0, er1, er2, eb1, ew2, eb2, ew3,
                     eb3, elg, elb, w1c, b1, w2, b2, w3, b3, lg, lb):
    E = gr.shape[0]
    grid = (E // BE,)

    def kfn(pr_r, ps_r, gr_r, gs_r, er0_r, er1_r, er2_r, eb1_r, ew2_r, eb2_r,
            ew3_r, eb3_r, elg_r, elb_r, w1_r, b1_r, w2_r, b2_r, w3_r, b3_r,
            lg_r, lb_r, en_o, eu_o):
        dx = (ps_r[:, 0:1] - pr_r[:, 0:1]) / RADIUS
        dy = (ps_r[:, 1:2] - pr_r[:, 1:2]) / RADIUS
        dist = jnp.sqrt(dx * dx + dy * dy)
        h = dx * er0_r[...] + dy * er1_r[...] + dist * er2_r[...] + eb1_r[...]
        h = jnp.maximum(h, 0.0)
        h = jnp.maximum(_dot(h, ew2_r[...]) + eb2_r[...], 0.0)
        e0 = _ln(_dot(h, ew3_r[...]) + eb3_r[...], elg_r[...], elb_r[...])
        h = gr_r[...] + gs_r[...] + _dot(e0, w1_r[...]) + b1_r[...]
        h = jnp.maximum(h, 0.0)
        h = jnp.maximum(_dot(h, w2_r[...]) + b2_r[...], 0.0)
        un = _ln(_dot(h, w3_r[...]) + b3_r[...], lg_r[...], lb_r[...])
        eu_o[...] = un
        en_o[...] = e0 + un

    return pl.pallas_call(
        kfn, grid=grid,
        in_specs=[_eblk(), _eblk(), _eblk(), _eblk(),
                  _B, _B, _B, _B, _W, _B, _W, _B, _B, _B,
                  _W, _B, _W, _B, _W, _B, _B, _B],
        out_specs=[_eblk(), _eblk()],
        out_shape=[jax.ShapeDtypeStruct((E, LATENT), _f32)] * 2,
    )(pr8, ps8, gr, gs, er0, er1, er2, eb1, ew2, eb2, ew3, eb3, elg, elb,
      w1c, b1, w2, b2, w3, b3, lg, lb)


def _edge_step_call(gr, gs, e, w1c, b1, w2, b2, w3, b3, lg, lb):
    E = gr.shape[0]
    grid = (E // BE,)

    def kfn(gr_r, gs_r, e_r, w1_r, b1_r, w2_r, b2_r, w3_r, b3_r, lg_r, lb_r,
            en_o, eu_o):
        e_blk = e_r[...]
        h = gr_r[...] + gs_r[...] + _dot(e_blk, w1_r[...]) + b1_r[...]
        h = jnp.maximum(h, 0.0)
        h = jnp.maximum(_dot(h, w2_r[...]) + b2_r[...], 0.0)
        un = _ln(_dot(h, w3_r[...]) + b3_r[...], lg_r[...], lb_r[...])
        eu_o[...] = un
        en_o[...] = e_blk + un

    return pl.pallas_call(
        kfn, grid=grid,
        in_specs=[_eblk(), _eblk(), _eblk(),
                  _W, _B, _W, _B, _W, _B, _B, _B],
        out_specs=[_eblk(), _eblk()],
        out_shape=[jax.ShapeDtypeStruct((E, LATENT), _f32)] * 2,
    )(gr, gs, e, w1c, b1, w2, b2, w3, b3, lg, lb)


def _node_body(x_r, agg_r, wa_r, wb_r, b1_r, w2_r, b2_r, w3_r, b3_r, lg_r, lb_r):
    x = x_r[...]
    ag = agg_r[0]
    for c in range(1, agg_r.shape[0]):
        ag = ag + agg_r[c]
    h = jnp.maximum(_dot(x, wa_r[...]) + _dot(ag, wb_r[...]) + b1_r[...], 0.0)
    h = jnp.maximum(_dot(h, w2_r[...]) + b2_r[...], 0.0)
    u = _dot(h, w3_r[...]) + b3_r[...]
    return x + _ln(u, lg_r[...], lb_r[...])


def _node_step_call(x, agg, wa, wb, b1, w2, b2, w3, b3, lg, lb, wra, wrb):
    nc = agg.shape[0]
    grid = (N_PAD // BN,)
    aspec = pl.BlockSpec((nc, BN, LATENT), lambda i: (0, i, 0))

    def kfn(x_r, agg_r, wa_r, wb_r, b1_r, w2_r, b2_r, w3_r, b3_r, lg_r, lb_r,
            wra_r, wrb_r, xn_o, xr_o, xs_o):
        xn = _node_body(x_r, agg_r, wa_r, wb_r, b1_r, w2_r, b2_r, w3_r, b3_r,
                        lg_r, lb_r)
        xn_o[...] = xn
        xr_o[...] = _dot(xn, wra_r[...])
        xs_o[...] = _dot(xn, wrb_r[...])

    return pl.pallas_call(
        kfn, grid=grid,
        in_specs=[_nblk(), aspec, _W, _W, _B, _W, _B, _W, _B, _B, _B, _W, _W],
        out_specs=[_nblk(), _nblk(), _nblk()],
        out_shape=[jax.ShapeDtypeStruct((N_PAD, LATENT), _f32)] * 3,
    )(x, agg, wa, wb, b1, w2, b2, w3, b3, lg, lb, wra, wrb)


def _node_final_call(x, agg, wa, wb, b1, w2, b2, w3, b3, lg, lb,
                     d1, db1, d2, db2, d3, db3, np8, nl8, yl8, yp8):
    nc = agg.shape[0]
    grid = (N_PAD // BN,)
    aspec = pl.BlockSpec((nc, BN, LATENT), lambda i: (0, i, 0))
    b8 = pl.BlockSpec((1, 8), lambda i: (0, 0))

    def kfn(x_r, agg_r, wa_r, wb_r, b1_r, w2_r, b2_r, w3_r, b3_r, lg_r, lb_r,
            d1_r, db1_r, d2_r, db2_r, d3_r, db3_r, np_r, nl_r, yl_r, yp_r,
            pred_o, tacc_o):
        xn = _node_body(x_r, agg_r, wa_r, wb_r, b1_r, w2_r, b2_r, w3_r, b3_r,
                        lg_r, lb_r)
        h = jnp.maximum(_dot(xn, d1_r[...]) + db1_r[...], 0.0)
        h = jnp.maximum(_dot(h, d2_r[...]) + db2_r[...], 0.0)
        pred_o[...] = _dot(h, d3_r[...]) + db3_r[...]
        tacc_o[...] = (np_r[...] + nl_r[...] - 2.0 * yl_r[...] + yp_r[...]) / ACC_STD

    return pl.pallas_call(
        kfn, grid=grid,
        in_specs=[_nblk(), aspec, _W, _W, _B, _W, _B, _W, _B, _B, _B,
                  _W, _B, _W, _B,
                  pl.BlockSpec((LATENT, 8), lambda i: (0, 0)), b8,
                  _nblk(8), _nblk(8), _nblk(8), _nblk(8)],
        out_specs=[_nblk(8), _nblk(8)],
        out_shape=[jax.ShapeDtypeStruct((N_PAD, 8), _f32)] * 2,
    )(x, agg, wa, wb, b1, w2, b2, w3, b3, lg, lb,
      d1, db1, d2, db2, d3, db3, np8, nl8, yl8, yp8)


def _row(v):
    return v.reshape(1, LATENT)


def kernel(next_positions, position_sequence_noise, position_sequence,
           nparticles_per_example, particle_types, params):
    N = position_sequence.shape[0]
    noisy = position_sequence + position_sequence_noise
    pos = noisy[:, -1]

    recv, send = _radius_edges(pos, nparticles_per_example)
    pad_e = jnp.full((E_PAD - MAX_EDGES,), N, jnp.int32)
    recv_p = jnp.concatenate([recv, pad_e])
    send_p = jnp.concatenate([send, pad_e])

    pn = N_PAD - N
    seq12 = jnp.pad(noisy.reshape(N, SEQ_LEN * PDIM), ((0, pn), (0, 0)))
    ty = jnp.pad(particle_types.astype(jnp.int32), (0, pn)).reshape(N_PAD, 1)
    pos128 = jnp.pad(pos, ((0, pn), (0, LATENT - PDIM)))
    np8 = jnp.pad(next_positions, ((0, pn), (0, 8 - PDIM)))
    nl8 = jnp.pad(position_sequence_noise[:, -1], ((0, pn), (0, 8 - PDIM)))
    yl8 = jnp.pad(pos, ((0, pn), (0, 8 - PDIM)))
    yp8 = jnp.pad(noisy[:, -2], ((0, pn), (0, 8 - PDIM)))
    zeros_agg = jnp.zeros((N_PAD, LATENT), _f32)

    temb = jnp.pad(params["type_emb"], ((0, 16 - NTYPES), (0, 0)))
    ne = params["node_enc"]
    wn1 = ne[0]["w"]
    ee = params["edge_enc"]
    ew1 = ee[0]["w"]
    dec = params["decoder"]
    d3 = jnp.pad(dec[2]["w"], ((0, 0), (0, 8 - PDIM)))
    db3 = jnp.pad(dec[2]["b"], (0, 8 - PDIM)).reshape(1, 8)

    steps = []
    for sp in params["steps"]:
        em, nm = sp["edge_mlp"], sp["node_mlp"]
        steps.append(dict(
            w1a=em[0]["w"][:LATENT], w1b=em[0]["w"][LATENT:2 * LATENT],
            w1c=em[0]["w"][2 * LATENT:], b1=_row(em[0]["b"]),
            w2=em[1]["w"], b2=_row(em[1]["b"]),
            w3=em[2]["w"], b3=_row(em[2]["b"]),
            lg=_row(sp["edge_ln"]["g"]), lb=_row(sp["edge_ln"]["b"]),
            nwa=nm[0]["w"][:LATENT], nwb=nm[0]["w"][LATENT:],
            nb1=_row(nm[0]["b"]), nw2=nm[1]["w"], nb2=_row(nm[1]["b"]),
            nw3=nm[2]["w"], nb3=_row(nm[2]["b"]),
            nlg=_row(sp["node_ln"]["g"]), nlb=_row(sp["node_ln"]["b"]),
        ))

    pr, ps = _sc_gather_pair(pos128, pos128, recv_p, send_p)

    x, xr, xs = _node_enc_call(
        seq12, ty, temb, wn1[0:10], wn1[10:14], wn1[14:30], _row(ne[0]["b"]),
        ne[1]["w"], _row(ne[1]["b"]), ne[2]["w"], _row(ne[2]["b"]),
        _row(params["node_enc_ln"]["g"]), _row(params["node_enc_ln"]["b"]),
        steps[0]["w1a"], steps[0]["w1b"])

    e = None
    for s in range(NSTEPS):
        st = steps[s]
        gr, gs = _sc_gather_pair(xr, xs, recv_p, send_p)
        if s == 0:
            e, eu = _edge_step0_call(
                pr, ps, gr, gs,
                ew1[0:1], ew1[1:2], ew1[2:3], _row(ee[0]["b"]),
                ee[1]["w"], _row(ee[1]["b"]), ee[2]["w"], _row(ee[2]["b"]),
                _row(params["edge_enc_ln"]["g"]), _row(params["edge_enc_ln"]["b"]),
                st["w1c"], st["b1"], st["w2"], st["b2"], st["w3"], st["b3"],
                st["lg"], st["lb"])
        else:
            e, eu = _edge_step_call(gr, gs, e, st["w1c"], st["b1"], st["w2"],
                                    st["b2"], st["w3"], st["b3"], st["lg"],
                                    st["lb"])
        agg = _sc_scatter_add(eu, recv_p, zeros_agg)
        if s < NSTEPS - 1:
            nx = steps[s + 1]
            x, xr, xs = _node_step_call(
                x, agg, st["nwa"], st["nwb"], st["nb1"], st["nw2"], st["nb2"],
                st["nw3"], st["nb3"], st["nlg"], st["nlb"],
                nx["w1a"], nx["w1b"])
        else:
            pred8, tacc8 = _node_final_call(
                x, agg, st["nwa"], st["nwb"], st["nb1"], st["nw2"], st["nb2"],
                st["nw3"], st["nb3"], st["nlg"], st["nlb"],
                dec[0]["w"], _row(dec[0]["b"]), dec[1]["w"], _row(dec[1]["b"]),
                d3, db3, np8, nl8, yl8, yp8)

    return pred8[:N, :PDIM], tacc8[:N, :PDIM]


# speedup vs baseline: 3.6129x; 3.6129x over previous
import functools

import jax
import jax.numpy as jnp
import numpy as np
from jax import lax
from jax.experimental import pallas as pl
from jax.experimental.pallas import tpu as pltpu
from jax.experimental.pallas import tpu_sc as plsc

PDIM = 2
SEQ_LEN = 6
LATENT = 128
NSTEPS = 5
RADIUS = 0.032
NTYPES = 9
CLAMP = 1.0
VEL_STD = 0.0017
ACC_STD = 3e-4
MAX_EDGES = 600000

N_PAD = 10240
E_PAD = 606208
BN = 512
BE = 1024
SC_CHUNK = 128

_f32 = jnp.float32


def _radius_edges(pos, nper):
    n_total = pos.shape[0]
    r2 = (RADIUS + 1e-8) ** 2
    thr = np.float32(r2)
    if float(thr) > r2:
        thr = np.nextafter(thr, np.float32(0.0))
    csum = jnp.cumsum(nper)
    ex = jnp.searchsorted(csum, jnp.arange(n_total), side="right")
    valid = ex < nper.shape[0]
    rows = []
    for i0 in range(0, n_total, 2048):
        blk = pos[i0:i0 + 2048]
        d2 = ((blk[:, None, :] - pos[None, :, :]) ** 2).sum(-1)
        m = (d2 <= thr) & (ex[i0:i0 + 2048, None] == ex[None, :]) & valid[i0:i0 + 2048, None]
        rows.append(m)
    mask = jnp.concatenate(rows, axis=0)
    receivers, senders = jnp.nonzero(mask, size=MAX_EDGES, fill_value=n_total)
    return receivers.astype(jnp.int32), senders.astype(jnp.int32)


_GR = 6


def _sc_gather(table, idx):
    E = idx.shape[0]
    D = table.shape[1]
    info = plsc.get_sparse_core_info()
    nc, ns = info.num_cores, info.num_subcores
    nw = nc * ns
    bw = E // nw
    nch = bw // SC_CHUNK
    mesh = plsc.VectorSubcoreMesh(core_axis_name="c", subcore_axis_name="s")

    scratch = ([pltpu.VMEM((SC_CHUNK,), jnp.int32) for _ in range(_GR)]
               + [pltpu.VMEM((SC_CHUNK, D), _f32) for _ in range(_GR)]
               + [pltpu.SemaphoreType.DMA for _ in range(3 * _GR)])

    @functools.partial(
        pl.kernel, mesh=mesh,
        out_type=jax.ShapeDtypeStruct((E, D), _f32),
        scratch_types=scratch,
    )
    def k(t_h, i_h, o_h, *bufs):
        ivs = bufs[:_GR]
        rvs = bufs[_GR:2 * _GR]
        sis = bufs[2 * _GR:3 * _GR]
        sgs = bufs[3 * _GR:4 * _GR]
        sws = bufs[4 * _GR:5 * _GR]
        wid = lax.axis_index("s") * nc + lax.axis_index("c")
        base = wid * bw
        hi, hg, hw = {}, {}, {}
        for t in range(nch + 5):
            ja = t
            if ja < nch:
                r = ja % _GR
                if ja >= _GR:
                    hw.pop(ja - _GR).wait()
                hi[ja] = pltpu.async_copy(
                    i_h.at[pl.ds(base + ja * SC_CHUNK, SC_CHUNK)], ivs[r], sis[r])
            jb = t - 1
            if 0 <= jb < nch:
                r = jb % _GR
                hi.pop(jb).wait()
                hg[jb] = pltpu.async_copy(t_h.at[ivs[r]], rvs[r], sgs[r])
            jc = t - 4
            if 0 <= jc < nch:
                r = jc % _GR
                hg.pop(jc).wait()
                hw[jc] = pltpu.async_copy(
                    rvs[r], o_h.at[pl.ds(base + jc * SC_CHUNK, SC_CHUNK)], sws[r])
        for j in sorted(hw):
            hw[j].wait()

    return k(table, idx)


_SR = 3
_SCC = 64


def _sc_scatter_add(rows, idx, zeros):
    E = idx.shape[0]
    info = plsc.get_sparse_core_info()
    nc, ns = info.num_cores, info.num_subcores
    nw = nc * ns
    bw = E // nw
    nch = bw // _SCC
    rps = N_PAD // ns
    mesh = plsc.VectorSubcoreMesh(core_axis_name="c", subcore_axis_name="s")

    @functools.partial(
        pl.kernel, mesh=mesh,
        out_type=jax.ShapeDtypeStruct((nc, N_PAD, LATENT), _f32),
        scratch_types=(
            [pltpu.VMEM((_SCC,), jnp.int32) for _ in range(_SR)]
            + [pltpu.VMEM((_SCC, LATENT), _f32) for _ in range(_SR)]
            + [pltpu.SemaphoreType.DMA for _ in range(3 * _SR)]
            + [pltpu.VMEM_SHARED((N_PAD, LATENT), _f32)]),
    )
    def k(rows_h, idx_h, zeros_h, out_h, *bufs):
        ivs = bufs[:_SR]
        rvs = bufs[_SR:2 * _SR]
        sis = bufs[2 * _SR:3 * _SR]
        srs = bufs[3 * _SR:4 * _SR]
        sss = bufs[4 * _SR:5 * _SR]
        acc = bufs[5 * _SR]
        cid = lax.axis_index("c")
        sid = lax.axis_index("s")
        wid = sid * nc + cid
        base = wid * bw
        pltpu.sync_copy(zeros_h.at[pl.ds(sid * rps, rps)],
                        acc.at[pl.ds(sid * rps, rps)])
        plsc.subcore_barrier()
        hi, hr, hs = {}, {}, {}
        for t in range(nch + 2):
            ja = t
            if ja < nch:
                r = ja % _SR
                if ja >= _SR:
                    hs.pop(ja - _SR).wait()
                off = base + ja * _SCC
                hi[ja] = pltpu.async_copy(
                    idx_h.at[pl.ds(off, _SCC)], ivs[r], sis[r])
                hr[ja] = pltpu.async_copy(
                    rows_h.at[pl.ds(off, _SCC)], rvs[r], srs[r])
            jb = t - 2
            if 0 <= jb < nch:
                r = jb % _SR
                hi.pop(jb).wait()
                hr.pop(jb).wait()
                hs[jb] = pltpu.async_copy(rvs[r], acc.at[ivs[r]], sss[r],
                                          add=True)
        for j in sorted(hs):
            hs[j].wait()
        plsc.subcore_barrier()
        pltpu.sync_copy(acc.at[pl.ds(sid * rps, rps)],
                        out_h.at[cid, pl.ds(sid * rps, rps)])

    return k(rows, idx, zeros)


def _ln(u, g, b):
    mu = jnp.mean(u, axis=-1, keepdims=True)
    var = jnp.mean((u - mu) ** 2, axis=-1, keepdims=True)
    return (u - mu) * lax.rsqrt(var + 1e-5) * g + b


def _dot(a, b):
    return jnp.dot(a, b, preferred_element_type=_f32)


_W = pl.BlockSpec((LATENT, LATENT), lambda i: (0, 0))
_B = pl.BlockSpec((1, LATENT), lambda i: (0, 0))


def _eblk(d=LATENT):
    return pl.BlockSpec((BE, d), lambda i: (i, 0))


def _nblk(d=LATENT):
    return pl.BlockSpec((BN, d), lambda i: (i, 0))


def _node_enc_call(seq12, ty, temb, w1v, w1bd, w1e, b1, w2, b2, w3, b3,
                   lg, lb, wra, wrb):
    grid = (N_PAD // BN,)

    def kfn(seq_r, ty_r, temb_r, w1v_r, w1bd_r, w1e_r, b1_r, w2_r, b2_r,
            w3_r, b3_r, lg_r, lb_r, wra_r, wrb_r, x_o, xr_o, xs_o):
        a = seq_r[...]
        v = (a[:, 2:12] - a[:, 0:10]) / VEL_STD
        mr = a[:, 10:12]
        bd = jnp.clip(
            jnp.concatenate([mr / RADIUS, (1.0 - mr) / RADIUS], axis=1),
            -CLAMP, CLAMP)
        t = ty_r[...]
        lanes = lax.broadcasted_iota(jnp.int32, (BN, 16), 1)
        oh = (lanes == t).astype(_f32)
        emb = _dot(oh, temb_r[...])
        h = _dot(v, w1v_r[...]) + _dot(bd, w1bd_r[...]) + _dot(emb, w1e_r[...]) + b1_r[...]
        h = jnp.maximum(h, 0.0)
        h = jnp.maximum(_dot(h, w2_r[...]) + b2_r[...], 0.0)
        u = _dot(h, w3_r[...]) + b3_r[...]
        x0 = _ln(u, lg_r[...], lb_r[...])
        x_o[...] = x0
        xr_o[...] = _dot(x0, wra_r[...])
        xs_o[...] = _dot(x0, wrb_r[...])

    return pl.pallas_call(
        kfn, grid=grid,
        in_specs=[
            _nblk(12),
            pl.BlockSpec((BN, 1), lambda i: (i, 0)),
            pl.BlockSpec((16, 16), lambda i: (0, 0)),
            pl.BlockSpec((10, LATENT), lambda i: (0, 0)),
            pl.BlockSpec((4, LATENT), lambda i: (0, 0)),
            pl.BlockSpec((16, LATENT), lambda i: (0, 0)),
            _B, _W, _B, _W, _B, _B, _B, _W, _W,
        ],
        out_specs=[_nblk(), _nblk(), _nblk()],
        out_shape=[jax.ShapeDtypeStruct((N_PAD, LATENT), _f32)] * 3,
    )(seq12, ty, temb, w1v, w1bd, w1e, b1, w2, b2, w3, b3, lg, lb, wra, wrb)


def _edge_step0_call(pr8, ps8, gr, gs, er0, er1, er2, eb1, ew2, eb2, ew3,
                     eb3, elg, elb, w1c, b1, w2, b2, w3, b3, lg, lb):
    E = gr.shape[0]
    grid = (E // BE,)

    def kfn(pr_r, ps_r, gr_r, gs_r, er0_r, er1_r, er2_r, eb1_r, ew2_r, eb2_r,
            ew3_r, eb3_r, elg_r, elb_r, w1_r, b1_r, w2_r, b2_r, w3_r, b3_r,
            lg_r, lb_r, en_o, eu_o):
        dx = (ps_r[:, 0:1] - pr_r[:, 0:1]) / RADIUS
        dy = (ps_r[:, 1:2] - pr_r[:, 1:2]) / RADIUS
        dist = jnp.sqrt(dx * dx + dy * dy)
        h = dx * er0_r[...] + dy * er1_r[...] + dist * er2_r[...] + eb1_r[...]
        h = jnp.maximum(h, 0.0)
        h = jnp.maximum(_dot(h, ew2_r[...]) + eb2_r[...], 0.0)
        e0 = _ln(_dot(h, ew3_r[...]) + eb3_r[...], elg_r[...], elb_r[...])
        h = gr_r[...] + gs_r[...] + _dot(e0, w1_r[...]) + b1_r[...]
        h = jnp.maximum(h, 0.0)
        h = jnp.maximum(_dot(h, w2_r[...]) + b2_r[...], 0.0)
        un = _ln(_dot(h, w3_r[...]) + b3_r[...], lg_r[...], lb_r[...])
        eu_o[...] = un
        en_o[...] = e0 + un

    return pl.pallas_call(
        kfn, grid=grid,
        in_specs=[_eblk(), _eblk(), _eblk(), _eblk(),
                  _B, _B, _B, _B, _W, _B, _W, _B, _B, _B,
                  _W, _B, _W, _B, _W, _B, _B, _B],
        out_specs=[_eblk(), _eblk()],
        out_shape=[jax.ShapeDtypeStruct((E, LATENT), _f32)] * 2,
    )(pr8, ps8, gr, gs, er0, er1, er2, eb1, ew2, eb2, ew3, eb3, elg, elb,
      w1c, b1, w2, b2, w3, b3, lg, lb)


def _edge_step_call(gr, gs, e, w1c, b1, w2, b2, w3, b3, lg, lb):
    E = gr.shape[0]
    grid = (E // BE,)

    def kfn(gr_r, gs_r, e_r, w1_r, b1_r, w2_r, b2_r, w3_r, b3_r, lg_r, lb_r,
            en_o, eu_o):
        e_blk = e_r[...]
        h = gr_r[...] + gs_r[...] + _dot(e_blk, w1_r[...]) + b1_r[...]
        h = jnp.maximum(h, 0.0)
        h = jnp.maximum(_dot(h, w2_r[...]) + b2_r[...], 0.0)
        un = _ln(_dot(h, w3_r[...]) + b3_r[...], lg_r[...], lb_r[...])
        eu_o[...] = un
        en_o[...] = e_blk + un

    return pl.pallas_call(
        kfn, grid=grid,
        in_specs=[_eblk(), _eblk(), _eblk(),
                  _W, _B, _W, _B, _W, _B, _B, _B],
        out_specs=[_eblk(), _eblk()],
        out_shape=[jax.ShapeDtypeStruct((E, LATENT), _f32)] * 2,
    )(gr, gs, e, w1c, b1, w2, b2, w3, b3, lg, lb)


def _node_body(x_r, agg_r, wa_r, wb_r, b1_r, w2_r, b2_r, w3_r, b3_r, lg_r, lb_r):
    x = x_r[...]
    ag = agg_r[0]
    for c in range(1, agg_r.shape[0]):
        ag = ag + agg_r[c]
    h = jnp.maximum(_dot(x, wa_r[...]) + _dot(ag, wb_r[...]) + b1_r[...], 0.0)
    h = jnp.maximum(_dot(h, w2_r[...]) + b2_r[...], 0.0)
    u = _dot(h, w3_r[...]) + b3_r[...]
    return x + _ln(u, lg_r[...], lb_r[...])


def _node_step_call(x, agg, wa, wb, b1, w2, b2, w3, b3, lg, lb, wra, wrb):
    nc = agg.shape[0]
    grid = (N_PAD // BN,)
    aspec = pl.BlockSpec((nc, BN, LATENT), lambda i: (0, i, 0))

    def kfn(x_r, agg_r, wa_r, wb_r, b1_r, w2_r, b2_r, w3_r, b3_r, lg_r, lb_r,
            wra_r, wrb_r, xn_o, xr_o, xs_o):
        xn = _node_body(x_r, agg_r, wa_r, wb_r, b1_r, w2_r, b2_r, w3_r, b3_r,
                        lg_r, lb_r)
        xn_o[...] = xn
        xr_o[...] = _dot(xn, wra_r[...])
        xs_o[...] = _dot(xn, wrb_r[...])

    return pl.pallas_call(
        kfn, grid=grid,
        in_specs=[_nblk(), aspec, _W, _W, _B, _W, _B, _W, _B, _B, _B, _W, _W],
        out_specs=[_nblk(), _nblk(), _nblk()],
        out_shape=[jax.ShapeDtypeStruct((N_PAD, LATENT), _f32)] * 3,
    )(x, agg, wa, wb, b1, w2, b2, w3, b3, lg, lb, wra, wrb)


def _node_final_call(x, agg, wa, wb, b1, w2, b2, w3, b3, lg, lb,
                     d1, db1, d2, db2, d3, db3, np8, nl8, yl8, yp8):
    nc = agg.shape[0]
    grid = (N_PAD // BN,)
    aspec = pl.BlockSpec((nc, BN, LATENT), lambda i: (0, i, 0))
    b8 = pl.BlockSpec((1, 8), lambda i: (0, 0))

    def kfn(x_r, agg_r, wa_r, wb_r, b1_r, w2_r, b2_r, w3_r, b3_r, lg_r, lb_r,
            d1_r, db1_r, d2_r, db2_r, d3_r, db3_r, np_r, nl_r, yl_r, yp_r,
            pred_o, tacc_o):
        xn = _node_body(x_r, agg_r, wa_r, wb_r, b1_r, w2_r, b2_r, w3_r, b3_r,
                        lg_r, lb_r)
        h = jnp.maximum(_dot(xn, d1_r[...]) + db1_r[...], 0.0)
        h = jnp.maximum(_dot(h, d2_r[...]) + db2_r[...], 0.0)
        pred_o[...] = _dot(h, d3_r[...]) + db3_r[...]
        tacc_o[...] = (np_r[...] + nl_r[...] - 2.0 * yl_r[...] + yp_r[...]) / ACC_STD

    return pl.pallas_call(
        kfn, grid=grid,
        in_specs=[_nblk(), aspec, _W, _W, _B, _W, _B, _W, _B, _B, _B,
                  _W, _B, _W, _B,
                  pl.BlockSpec((LATENT, 8), lambda i: (0, 0)), b8,
                  _nblk(8), _nblk(8), _nblk(8), _nblk(8)],
        out_specs=[_nblk(8), _nblk(8)],
        out_shape=[jax.ShapeDtypeStruct((N_PAD, 8), _f32)] * 2,
    )(x, agg, wa, wb, b1, w2, b2, w3, b3, lg, lb,
      d1, db1, d2, db2, d3, db3, np8, nl8, yl8, yp8)


def _row(v):
    return v.reshape(1, LATENT)


def kernel(next_positions, position_sequence_noise, position_sequence,
           nparticles_per_example, particle_types, params):
    N = position_sequence.shape[0]
    noisy = position_sequence + position_sequence_noise
    pos = noisy[:, -1]

    recv, send = _radius_edges(pos, nparticles_per_example)
    pad_e = jnp.full((E_PAD - MAX_EDGES,), N, jnp.int32)
    recv_p = jnp.concatenate([recv, pad_e])
    send_p = jnp.concatenate([send, pad_e])

    pn = N_PAD - N
    seq12 = jnp.pad(noisy.reshape(N, SEQ_LEN * PDIM), ((0, pn), (0, 0)))
    ty = jnp.pad(particle_types.astype(jnp.int32), (0, pn)).reshape(N_PAD, 1)
    pos128 = jnp.pad(pos, ((0, pn), (0, LATENT - PDIM)))
    np8 = jnp.pad(next_positions, ((0, pn), (0, 8 - PDIM)))
    nl8 = jnp.pad(position_sequence_noise[:, -1], ((0, pn), (0, 8 - PDIM)))
    yl8 = jnp.pad(pos, ((0, pn), (0, 8 - PDIM)))
    yp8 = jnp.pad(noisy[:, -2], ((0, pn), (0, 8 - PDIM)))
    zeros_agg = jnp.zeros((N_PAD, LATENT), _f32)

    temb = jnp.pad(params["type_emb"], ((0, 16 - NTYPES), (0, 0)))
    ne = params["node_enc"]
    wn1 = ne[0]["w"]
    ee = params["edge_enc"]
    ew1 = ee[0]["w"]
    dec = params["decoder"]
    d3 = jnp.pad(dec[2]["w"], ((0, 0), (0, 8 - PDIM)))
    db3 = jnp.pad(dec[2]["b"], (0, 8 - PDIM)).reshape(1, 8)

    steps = []
    for sp in params["steps"]:
        em, nm = sp["edge_mlp"], sp["node_mlp"]
        steps.append(dict(
            w1a=em[0]["w"][:LATENT], w1b=em[0]["w"][LATENT:2 * LATENT],
            w1c=em[0]["w"][2 * LATENT:], b1=_row(em[0]["b"]),
            w2=em[1]["w"], b2=_row(em[1]["b"]),
            w3=em[2]["w"], b3=_row(em[2]["b"]),
            lg=_row(sp["edge_ln"]["g"]), lb=_row(sp["edge_ln"]["b"]),
            nwa=nm[0]["w"][:LATENT], nwb=nm[0]["w"][LATENT:],
            nb1=_row(nm[0]["b"]), nw2=nm[1]["w"], nb2=_row(nm[1]["b"]),
            nw3=nm[2]["w"], nb3=_row(nm[2]["b"]),
            nlg=_row(sp["node_ln"]["g"]), nlb=_row(sp["node_ln"]["b"]),
        ))

    x0, xr0, xs0 = _node_enc_call(
        seq12, ty, temb, wn1[0:10], wn1[10:14], wn1[14:30], _row(ne[0]["b"]),
        ne[1]["w"], _row(ne[1]["b"]), ne[2]["w"], _row(ne[2]["b"]),
        _row(params["node_enc_ln"]["g"]), _row(params["node_enc_ln"]["b"]),
        steps[0]["w1a"], steps[0]["w1b"])

    def _mp_at(E):
        def fn(_):
            rp = lax.slice_in_dim(recv_p, 0, E)
            sp = lax.slice_in_dim(send_p, 0, E)
            pr = _sc_gather(pos128, rp)
            ps = _sc_gather(pos128, sp)
            x, xr, xs = x0, xr0, xs0
            e = pred8 = tacc8 = None
            for s in range(NSTEPS):
                st = steps[s]
                gr = _sc_gather(xr, rp)
                gs = _sc_gather(xs, sp)
                if s == 0:
                    e, eu = _edge_step0_call(
                        pr, ps, gr, gs,
                        ew1[0:1], ew1[1:2], ew1[2:3], _row(ee[0]["b"]),
                        ee[1]["w"], _row(ee[1]["b"]), ee[2]["w"], _row(ee[2]["b"]),
                        _row(params["edge_enc_ln"]["g"]),
                        _row(params["edge_enc_ln"]["b"]),
                        st["w1c"], st["b1"], st["w2"], st["b2"], st["w3"],
                        st["b3"], st["lg"], st["lb"])
                else:
                    e, eu = _edge_step_call(gr, gs, e, st["w1c"], st["b1"],
                                            st["w2"], st["b2"], st["w3"],
                                            st["b3"], st["lg"], st["lb"])
                agg = _sc_scatter_add(eu, rp, zeros_agg)
                if s < NSTEPS - 1:
                    nx = steps[s + 1]
                    x, xr, xs = _node_step_call(
                        x, agg, st["nwa"], st["nwb"], st["nb1"], st["nw2"],
                        st["nb2"], st["nw3"], st["nb3"], st["nlg"], st["nlb"],
                        nx["w1a"], nx["w1b"])
                else:
                    pred8, tacc8 = _node_final_call(
                        x, agg, st["nwa"], st["nwb"], st["nb1"], st["nw2"],
                        st["nb2"], st["nw3"], st["nb3"], st["nlg"], st["nlb"],
                        dec[0]["w"], _row(dec[0]["b"]), dec[1]["w"],
                        _row(dec[1]["b"]), d3, db3, np8, nl8, yl8, yp8)
            return pred8, tacc8
        return fn

    buckets = [229376, 311296, E_PAD]
    count = jnp.searchsorted(recv, jnp.int32(N), side="left")
    sel = ((count > buckets[0]).astype(jnp.int32)
           + (count > buckets[1]).astype(jnp.int32))
    pred8, tacc8 = lax.switch(sel, [_mp_at(b) for b in buckets], None)

    return pred8[:N, :PDIM], tacc8[:N, :PDIM]
